# baseline (speedup 1.0000x reference)
import functools

import jax
import jax.numpy as jnp
from jax import lax
from jax.experimental import pallas as pl
from jax.experimental.pallas import tpu as pltpu
from jax.experimental.pallas import tpu_sc as plsc

BATCH = 16384
EMBED_DIM = 64
VOCAB_PAIRS = 500000

_NC = 2
_NS = 16
_NW = _NC * _NS
_BPW = BATCH // _NW
_CHUNK = 128
_NCHUNK = _BPW // _CHUNK


def _gather_kernel(idx_hbm, table_hbm, out_hbm, idx_v, q_v, buf_v, out_v,
                   sem0, sem1):
    wid = lax.axis_index("s") * _NC + lax.axis_index("c")
    base = wid * _BPW
    sems = (sem0, sem1)
    pltpu.sync_copy(idx_hbm.at[wid], idx_v)
    for j in range(_NCHUNK):
        for k in range(_CHUNK // 16):
            v = idx_v[pl.ds(j * _CHUNK + k * 16, 16)] >> 1
            q_v[j, pl.ds(k * 16, 16)] = v

    iota = lax.iota(jnp.int32, 16)

    def fire(j):
        return pltpu.async_copy(
            table_hbm.at[q_v.at[j]], buf_v.at[j % 2], sems[j % 2])

    def extract(j, desc):
        desc.wait()
        buf = buf_v.at[j % 2]

        def body(g, carry):
            r0 = j * _CHUNK + g * 16
            rows = iota + g * 16
            h = (idx_v[pl.ds(r0, 16)] & 1) * 64
            orows = iota + r0
            for c in range(EMBED_DIM):
                val = plsc.load_gather(buf, [rows, h + c])
                plsc.store_scatter(
                    out_v, [orows, jnp.full((16,), c, jnp.int32)], val)
            return carry

        lax.fori_loop(0, _CHUNK // 16, body, 0)

    desc = fire(0)
    for j in range(_NCHUNK):
        nxt = fire(j + 1) if j + 1 < _NCHUNK else None
        extract(j, desc)
        desc = nxt

    pltpu.sync_copy(out_v, out_hbm.at[pl.ds(base, _BPW)])


@jax.jit
def _embed_gather(idx_r, table2):
    mesh = plsc.VectorSubcoreMesh(core_axis_name="c", subcore_axis_name="s")
    run = functools.partial(
        pl.kernel,
        mesh=mesh,
        out_type=jax.ShapeDtypeStruct((BATCH, EMBED_DIM), jnp.float32),
        scratch_types=[
            pltpu.VMEM((_BPW,), jnp.int32),
            pltpu.VMEM((_NCHUNK, _CHUNK), jnp.int32),
            pltpu.VMEM((2, _CHUNK, 2 * EMBED_DIM), jnp.float32),
            pltpu.VMEM((_BPW, EMBED_DIM), jnp.float32),
            pltpu.SemaphoreType.DMA,
            pltpu.SemaphoreType.DMA,
        ],
        compiler_params=pltpu.CompilerParams(needs_layout_passes=False),
    )(_gather_kernel)
    return run(idx_r, table2)


def kernel(in_idxs, embed_in):
    idx_r = in_idxs.astype(jnp.int32).reshape(_NW, _BPW)
    table2 = embed_in.reshape(VOCAB_PAIRS, 2 * EMBED_DIM)
    return _embed_gather(idx_r, table2)
